# baseline (device time: 21412 ns/iter reference)
import os

import jax
import jax.numpy as jnp
from jax import lax
from jax.experimental import pallas as pl
from jax.experimental.pallas import tpu as pltpu

_PHASES = int(os.environ.get("KERNEL_PHASES", "2"))

N_DEV = 16
M = 512
N = 512
ROWS = M // N_DEV


def kernel(A, B):
    def body(a_ref, b_ref, out_ref, partial_ref, rs_ref,
             send_sems, recv_sems, send_sems2, recv_sems2, ready_sems):
        my = lax.axis_index("i")

        if _PHASES >= 1:
            for d in range(1, N_DEV):
                tgt = (my + d) % N_DEV
                pl.semaphore_signal(
                    ready_sems.at[my], inc=1,
                    device_id=(tgt,), device_id_type=pl.DeviceIdType.MESH,
                )
        barrier_sem = pltpu.get_barrier_semaphore()
        pl.semaphore_signal(barrier_sem, inc=1)
        pl.semaphore_wait(barrier_sem, 1)

        a = a_ref[...].astype(jnp.bfloat16)
        b = b_ref[...].astype(jnp.bfloat16)
        partial_ref[...] = jnp.dot(
            a, b, preferred_element_type=jnp.float32
        ).astype(jnp.bfloat16)

        sends = []
        if _PHASES >= 1:
            for d in range(1, N_DEV):
                tgt = (my + d) % N_DEV
                pl.semaphore_wait(ready_sems.at[tgt], 1)
                rdma = pltpu.make_async_remote_copy(
                    src_ref=partial_ref.at[pl.ds(tgt * ROWS, ROWS), :],
                    dst_ref=rs_ref.at[my],
                    send_sem=send_sems.at[d - 1],
                    recv_sem=recv_sems.at[my],
                    device_id=(tgt,),
                    device_id_type=pl.DeviceIdType.MESH,
                )
                rdma.start()
                sends.append(rdma)

        rs_ref[pl.ds(my, 1)] = partial_ref[pl.ds(my * ROWS, ROWS), :][None]

        if _PHASES >= 1:
            for d in range(1, N_DEV):
                src = (my + d) % N_DEV
                recv = pltpu.make_async_remote_copy(
                    src_ref=partial_ref.at[pl.ds(0, ROWS), :],
                    dst_ref=rs_ref.at[src],
                    send_sem=send_sems.at[d - 1],
                    recv_sem=recv_sems.at[src],
                    device_id=(src,),
                    device_id_type=pl.DeviceIdType.MESH,
                )
                recv.wait_recv()

        z = jnp.sum(rs_ref[...].astype(jnp.float32), axis=0)
        g = 0.5 * z * (1.0 + jnp.tanh(0.7978845608 * (z + 0.044715 * z * z * z)))
        out_ref[pl.ds(my * ROWS, ROWS), :] = g.astype(jnp.bfloat16)

        sends2 = []
        if _PHASES >= 2:
            for d in range(1, N_DEV):
                tgt = (my + d) % N_DEV
                rdma = pltpu.make_async_remote_copy(
                    src_ref=out_ref.at[pl.ds(my * ROWS, ROWS), :],
                    dst_ref=out_ref.at[pl.ds(my * ROWS, ROWS), :],
                    send_sem=send_sems2.at[d - 1],
                    recv_sem=recv_sems2.at[my],
                    device_id=(tgt,),
                    device_id_type=pl.DeviceIdType.MESH,
                )
                rdma.start()
                sends2.append(rdma)

            for d in range(1, N_DEV):
                src = (my + d) % N_DEV
                recv = pltpu.make_async_remote_copy(
                    src_ref=out_ref.at[pl.ds(0, ROWS), :],
                    dst_ref=out_ref.at[pl.ds(src * ROWS, ROWS), :],
                    send_sem=send_sems2.at[d - 1],
                    recv_sem=recv_sems2.at[src],
                    device_id=(src,),
                    device_id_type=pl.DeviceIdType.MESH,
                )
                recv.wait_recv()

        for rdma in sends + sends2:
            rdma.wait_send()

    out_shape = jax.ShapeDtypeStruct((M, N), jnp.bfloat16)
    return pl.pallas_call(
        body,
        out_shape=out_shape,
        in_specs=[
            pl.BlockSpec(memory_space=pltpu.VMEM),
            pl.BlockSpec(memory_space=pltpu.VMEM),
        ],
        out_specs=pl.BlockSpec(memory_space=pltpu.VMEM),
        scratch_shapes=[
            pltpu.VMEM((M, N), jnp.bfloat16),
            pltpu.VMEM((N_DEV, ROWS, N), jnp.bfloat16),
            pltpu.SemaphoreType.DMA((N_DEV - 1,)),
            pltpu.SemaphoreType.DMA((N_DEV,)),
            pltpu.SemaphoreType.DMA((N_DEV - 1,)),
            pltpu.SemaphoreType.DMA((N_DEV,)),
            pltpu.SemaphoreType.REGULAR((N_DEV,)),
        ],
        compiler_params=pltpu.CompilerParams(collective_id=0),
    )(A, B)


# device time: 15053 ns/iter; 1.4224x vs baseline; 1.4224x over previous
import os

import jax
import jax.numpy as jnp
from jax import lax
from jax.experimental import pallas as pl
from jax.experimental.pallas import tpu as pltpu

_PHASES = int(os.environ.get("KERNEL_PHASES", "2"))

N_DEV = 16
M = 512
N = 512
ROWS = M // N_DEV


def kernel(A, B):
    def body(a_ref, b_ref, out_ref, partial_ref, rs_ref,
             send_sems, recv_sems, send_sems2, recv_sems2, ready_sems):
        my = lax.axis_index("i")

        if _PHASES >= 1:
            for d in range(1, N_DEV):
                tgt = (my + d) % N_DEV
                pl.semaphore_signal(
                    ready_sems.at[my], inc=1,
                    device_id=(tgt,), device_id_type=pl.DeviceIdType.MESH,
                )
        barrier_sem = pltpu.get_barrier_semaphore()
        pl.semaphore_signal(barrier_sem, inc=1)
        pl.semaphore_wait(barrier_sem, 1)

        b = b_ref[...].astype(jnp.bfloat16)

        sends = []
        if _PHASES >= 1:
            for d in range(1, N_DEV):
                tgt = (my + d) % N_DEV
                a_s = a_ref[pl.ds(tgt * ROWS, ROWS), :].astype(jnp.bfloat16)
                partial_ref[pl.ds(tgt * ROWS, ROWS), :] = jnp.dot(
                    a_s, b, preferred_element_type=jnp.float32
                ).astype(jnp.bfloat16)
                pl.semaphore_wait(ready_sems.at[tgt], 1)
                rdma = pltpu.make_async_remote_copy(
                    src_ref=partial_ref.at[pl.ds(tgt * ROWS, ROWS), :],
                    dst_ref=rs_ref.at[my],
                    send_sem=send_sems.at[d - 1],
                    recv_sem=recv_sems.at[my],
                    device_id=(tgt,),
                    device_id_type=pl.DeviceIdType.MESH,
                )
                rdma.start()
                sends.append(rdma)

        a_s = a_ref[pl.ds(my * ROWS, ROWS), :].astype(jnp.bfloat16)
        acc = jnp.dot(a_s, b, preferred_element_type=jnp.float32)

        if _PHASES >= 1:
            for d in range(1, N_DEV):
                src = (my + d) % N_DEV
                recv = pltpu.make_async_remote_copy(
                    src_ref=partial_ref.at[pl.ds(0, ROWS), :],
                    dst_ref=rs_ref.at[src],
                    send_sem=send_sems.at[d - 1],
                    recv_sem=recv_sems.at[src],
                    device_id=(src,),
                    device_id_type=pl.DeviceIdType.MESH,
                )
                recv.wait_recv()
                acc = acc + rs_ref[src].astype(jnp.float32)

        z = acc
        g = 0.5 * z * (1.0 + jnp.tanh(0.7978845608 * (z + 0.044715 * z * z * z)))
        out_ref[pl.ds(my * ROWS, ROWS), :] = g.astype(jnp.bfloat16)

        sends2 = []
        if _PHASES >= 2:
            for d in range(1, N_DEV):
                tgt = (my + d) % N_DEV
                rdma = pltpu.make_async_remote_copy(
                    src_ref=out_ref.at[pl.ds(my * ROWS, ROWS), :],
                    dst_ref=out_ref.at[pl.ds(my * ROWS, ROWS), :],
                    send_sem=send_sems2.at[d - 1],
                    recv_sem=recv_sems2.at[my],
                    device_id=(tgt,),
                    device_id_type=pl.DeviceIdType.MESH,
                )
                rdma.start()
                sends2.append(rdma)

            for d in range(1, N_DEV):
                src = (my + d) % N_DEV
                recv = pltpu.make_async_remote_copy(
                    src_ref=out_ref.at[pl.ds(0, ROWS), :],
                    dst_ref=out_ref.at[pl.ds(src * ROWS, ROWS), :],
                    send_sem=send_sems2.at[d - 1],
                    recv_sem=recv_sems2.at[src],
                    device_id=(src,),
                    device_id_type=pl.DeviceIdType.MESH,
                )
                recv.wait_recv()

        for rdma in sends + sends2:
            rdma.wait_send()

    out_shape = jax.ShapeDtypeStruct((M, N), jnp.bfloat16)
    return pl.pallas_call(
        body,
        out_shape=out_shape,
        in_specs=[
            pl.BlockSpec(memory_space=pltpu.VMEM),
            pl.BlockSpec(memory_space=pltpu.VMEM),
        ],
        out_specs=pl.BlockSpec(memory_space=pltpu.VMEM),
        scratch_shapes=[
            pltpu.VMEM((M, N), jnp.bfloat16),
            pltpu.VMEM((N_DEV, ROWS, N), jnp.bfloat16),
            pltpu.SemaphoreType.DMA((N_DEV - 1,)),
            pltpu.SemaphoreType.DMA((N_DEV,)),
            pltpu.SemaphoreType.DMA((N_DEV - 1,)),
            pltpu.SemaphoreType.DMA((N_DEV,)),
            pltpu.SemaphoreType.REGULAR((N_DEV,)),
        ],
        compiler_params=pltpu.CompilerParams(collective_id=0),
    )(A, B)


# device time: 14781 ns/iter; 1.4486x vs baseline; 1.0184x over previous
import os

import jax
import jax.numpy as jnp
from jax import lax
from jax.experimental import pallas as pl
from jax.experimental.pallas import tpu as pltpu

_PHASES = int(os.environ.get("KERNEL_PHASES", "2"))
_COMM_ONLY = int(os.environ.get("KERNEL_COMM_ONLY", "0"))

N_DEV = 16
M = 512
N = 512
ROWS = M // N_DEV


def kernel(A, B):
    def body(a_ref, b_ref, out_ref, partial_ref, rs_ref,
             send_sems, recv_sems, send_sems2, recv_sems2, ready_sems):
        my = lax.axis_index("i")

        if _PHASES >= 1:
            for d in range(1, N_DEV):
                tgt = (my + d) % N_DEV
                pl.semaphore_signal(
                    ready_sems.at[my], inc=1,
                    device_id=(tgt,), device_id_type=pl.DeviceIdType.MESH,
                )
        barrier_sem = pltpu.get_barrier_semaphore()
        pl.semaphore_signal(barrier_sem, inc=1)
        pl.semaphore_wait(barrier_sem, 1)

        b = b_ref[...].astype(jnp.bfloat16)

        sends = []
        if _PHASES >= 1:
            if _COMM_ONLY:
                partial_ref[...] = jnp.zeros((M, N), jnp.bfloat16)
            with jax.named_scope("rs_compute_send"):
                for d in range(1, N_DEV):
                    tgt = (my + d) % N_DEV
                    if not _COMM_ONLY:
                        a_s = a_ref[pl.ds(tgt * ROWS, ROWS), :].astype(jnp.bfloat16)
                        partial_ref[pl.ds(tgt * ROWS, ROWS), :] = jnp.dot(
                            a_s, b, preferred_element_type=jnp.float32
                        ).astype(jnp.bfloat16)
                    pl.semaphore_wait(ready_sems.at[tgt], 1)
                    rdma = pltpu.make_async_remote_copy(
                        src_ref=partial_ref.at[pl.ds(tgt * ROWS, ROWS), :],
                        dst_ref=rs_ref.at[my],
                        send_sem=send_sems.at[d - 1],
                        recv_sem=recv_sems.at[my],
                        device_id=(tgt,),
                        device_id_type=pl.DeviceIdType.MESH,
                    )
                    rdma.start()
                    sends.append(rdma)

        with jax.named_scope("own_stripe"):
            if _COMM_ONLY:
                acc = jnp.zeros((ROWS, N), jnp.float32)
            else:
                a_s = a_ref[pl.ds(my * ROWS, ROWS), :].astype(jnp.bfloat16)
                acc = jnp.dot(a_s, b, preferred_element_type=jnp.float32)

        if _PHASES >= 1:
            with jax.named_scope("rs_wait_accum"):
                for d in range(1, N_DEV):
                    src = (my + d) % N_DEV
                    recv = pltpu.make_async_remote_copy(
                        src_ref=partial_ref.at[pl.ds(0, ROWS), :],
                        dst_ref=rs_ref.at[src],
                        send_sem=send_sems.at[d - 1],
                        recv_sem=recv_sems.at[src],
                        device_id=(src,),
                        device_id_type=pl.DeviceIdType.MESH,
                    )
                    recv.wait_recv()
                    acc = acc + rs_ref[src].astype(jnp.float32)

        with jax.named_scope("gelu"):
            z = acc
            g = 0.5 * z * (1.0 + jnp.tanh(0.7978845608 * (z + 0.044715 * z * z * z)))
            out_ref[pl.ds(my * ROWS, ROWS), :] = g.astype(jnp.bfloat16)

        sends2 = []
        if _PHASES >= 2:
            with jax.named_scope("ag_send"):
                for d in range(1, N_DEV):
                    tgt = (my + d) % N_DEV
                    rdma = pltpu.make_async_remote_copy(
                        src_ref=out_ref.at[pl.ds(my * ROWS, ROWS), :],
                        dst_ref=out_ref.at[pl.ds(my * ROWS, ROWS), :],
                        send_sem=send_sems2.at[d - 1],
                        recv_sem=recv_sems2.at[my],
                        device_id=(tgt,),
                        device_id_type=pl.DeviceIdType.MESH,
                    )
                    rdma.start()
                    sends2.append(rdma)

            with jax.named_scope("ag_wait"):
                for d in range(1, N_DEV):
                    src = (my + d) % N_DEV
                    recv = pltpu.make_async_remote_copy(
                        src_ref=out_ref.at[pl.ds(0, ROWS), :],
                        dst_ref=out_ref.at[pl.ds(src * ROWS, ROWS), :],
                        send_sem=send_sems2.at[d - 1],
                        recv_sem=recv_sems2.at[src],
                        device_id=(src,),
                        device_id_type=pl.DeviceIdType.MESH,
                    )
                    recv.wait_recv()

        with jax.named_scope("drain_sends"):
            for rdma in sends + sends2:
                rdma.wait_send()

    out_shape = jax.ShapeDtypeStruct((M, N), jnp.bfloat16)
    return pl.pallas_call(
        body,
        out_shape=out_shape,
        in_specs=[
            pl.BlockSpec(memory_space=pltpu.VMEM),
            pl.BlockSpec(memory_space=pltpu.VMEM),
        ],
        out_specs=pl.BlockSpec(memory_space=pltpu.VMEM),
        scratch_shapes=[
            pltpu.VMEM((M, N), jnp.bfloat16),
            pltpu.VMEM((N_DEV, ROWS, N), jnp.bfloat16),
            pltpu.SemaphoreType.DMA((N_DEV - 1,)),
            pltpu.SemaphoreType.DMA((N_DEV,)),
            pltpu.SemaphoreType.DMA((N_DEV - 1,)),
            pltpu.SemaphoreType.DMA((N_DEV,)),
            pltpu.SemaphoreType.REGULAR((N_DEV,)),
        ],
        compiler_params=pltpu.CompilerParams(collective_id=0),
    )(A, B)
